# baseline (speedup 1.0000x reference)
import functools
import math

import jax
import jax.numpy as jnp
from jax import lax
from jax.experimental import pallas as pl
from jax.experimental.pallas import tpu as pltpu
from jax.experimental.pallas import tpu_sc as plsc

D_MODEL = 64
SCALE = math.sqrt(D_MODEL)

NC = 2
NS = 16
NW = NC * NS

IB = 128
L = 16
RING = 4


def _relayout_scale(tab_t):
    d, v = tab_t.shape
    assert d == D_MODEL
    W = 2
    n_full = v // IB
    n_bulk = (n_full // NW) * NW
    cols = n_bulk // NW
    steps = cols // W
    assert cols % W == 0
    n_tail = n_full - n_bulk
    v_rem = v - n_full * IB
    mesh = plsc.VectorSubcoreMesh(core_axis_name="c", subcore_axis_name="s")

    @functools.partial(
        pl.kernel,
        mesh=mesh,
        out_type=jax.ShapeDtypeStruct((v // 16, 8, IB), jnp.float32),
        scratch_types=[
            pltpu.VMEM((D_MODEL, W * IB), jnp.float32),
            pltpu.VMEM((D_MODEL, W * IB), jnp.float32),
            pltpu.VMEM((D_MODEL, IB), jnp.float32),
            pltpu.VMEM((8, 8, IB + 1), jnp.float32),
            pltpu.VMEM((8, 8, IB + 1), jnp.float32),
            pltpu.SemaphoreType.DMA,
            pltpu.SemaphoreType.DMA,
            pltpu.SemaphoreType.DMA,
            pltpu.SemaphoreType.DMA,
        ],
        compiler_params=pltpu.CompilerParams(
            use_tc_tiling_on_sc=True, needs_layout_passes=False
        ),
    )
    def ka(tab_hbm, tail_hbm, out_hbm, s0, s1, st, d0, d1, r0, r1, w0, w1):
        t = lax.axis_index("s") * NC + lax.axis_index("c")
        srcs = (s0, s1)
        dsts = (d0, d1)
        rsems = (r0, r1)
        wsems = (w0, w1)
        col0 = t * cols

        iota = lax.iota(jnp.int32, L)

        def fire_read(s, b):
            pltpu.async_copy(
                tab_hbm.at[:, pl.ds((col0 + s * W) * IB, W * IB)],
                srcs[b],
                rsems[b],
            )

        def wait_read(b):
            pltpu.make_async_copy(
                tab_hbm.at[:, pl.ds(0, W * IB)], srcs[b], rsems[b]
            ).wait()

        def dst_view(b):
            return dsts[b].at[:, :, pl.ds(0, IB)]

        def wait_wb(b):
            pltpu.make_async_copy(
                out_hbm.at[pl.ds(0, 8)], dst_view(b), wsems[b]
            ).wait()

        def transpose(src, base, d):
            for g in range(IB // L):
                u = iota + 16 * (g >> 1)
                uhi = u >> 3
                ulo = u & 7
                half = (g & 1) * D_MODEL

                @plsc.parallel_loop(0, D_MODEL, unroll=8)
                def _(c):
                    vals = src[c, pl.ds(base + g * L, L)]
                    off = jnp.full((L,), half + c, jnp.int32)
                    plsc.store_scatter(dsts[d], [uhi, ulo, off], vals * SCALE)

        fire_read(0, 0)

        def body(g2, carry):
            for b in range(2):
                s = g2 * 2 + b

                @pl.when(s + 1 < steps)
                def _():
                    fire_read(s + 1, 1 - b)

                wait_read(b)
                for w in range(W):
                    cc = s * W + w
                    d = w % 2

                    @pl.when(cc >= 2)
                    def _():
                        wait_wb(d)

                    transpose(srcs[b], w * IB, d)
                    pltpu.async_copy(
                        dst_view(d),
                        out_hbm.at[pl.ds((col0 + cc) * 8, 8)],
                        wsems[d],
                    )
            return carry

        lax.fori_loop(0, steps // 2, body, 0)
        wait_wb(0)
        wait_wb(1)

        @pl.when(t < n_tail + (1 if v_rem else 0))
        def _():
            rt = n_bulk + t

            @pl.when(t < n_tail)
            def _():
                pltpu.sync_copy(tab_hbm.at[:, pl.ds(rt * IB, IB)], st)

            @pl.when(t >= n_tail)
            def _():
                pltpu.sync_copy(tail_hbm, st)

            transpose(st, 0, 0)

            @pl.when(t < n_tail)
            def _():
                pltpu.sync_copy(dst_view(0), out_hbm.at[pl.ds(rt * 8, 8)])

            @pl.when(t >= n_tail)
            def _():
                pltpu.sync_copy(
                    dsts[0].at[pl.ds(0, v_rem // 16), :, pl.ds(0, IB)],
                    out_hbm.at[pl.ds(rt * 8, v_rem // 16)],
                )

    tail = jnp.pad(
        lax.slice(tab_t, (0, n_full * IB), (d, v)),
        ((0, 0), (0, IB - v_rem)),
    )
    return ka(tab_t, tail)


def _gather(xp, table):
    nj_hi, nt, nj_lo, ib = xp.shape
    nj = nj_hi * nj_lo
    assert nt == NW and ib == IB and nj % RING == 0
    mesh = plsc.VectorSubcoreMesh(core_axis_name="c", subcore_axis_name="s")

    @functools.partial(
        pl.kernel,
        mesh=mesh,
        out_type=jax.ShapeDtypeStruct(
            (nj, D_MODEL // 8, NW, 8, IB), jnp.float32
        ),
        scratch_types=[
            pltpu.VMEM((nj_hi, nj_lo, IB), jnp.int32),
            pltpu.VMEM((IB, D_MODEL), jnp.float32),
            pltpu.VMEM((IB, D_MODEL), jnp.float32),
            pltpu.VMEM((IB, D_MODEL), jnp.float32),
            pltpu.VMEM((IB, D_MODEL), jnp.float32),
            pltpu.VMEM((D_MODEL // 8, 8, IB + 1), jnp.float32),
            pltpu.VMEM((D_MODEL // 8, 8, IB + 1), jnp.float32),
            pltpu.SemaphoreType.DMA,
            pltpu.SemaphoreType.DMA,
            pltpu.SemaphoreType.DMA,
            pltpu.SemaphoreType.DMA,
            pltpu.SemaphoreType.DMA,
            pltpu.SemaphoreType.DMA,
        ],
        compiler_params=pltpu.CompilerParams(
            use_tc_tiling_on_sc=False, needs_layout_passes=False
        ),
    )
    def kb(xp_hbm, table_hbm, out_hbm,
           xv, r0, r1, r2, r3, b0, b1, g0, g1, g2, g3, w0, w1):
        t = lax.axis_index("s") * NC + lax.axis_index("c")
        rows = (r0, r1, r2, r3)
        blks = (b0, b1)
        gsems = (g0, g1, g2, g3)
        wsems = (w0, w1)

        pltpu.sync_copy(xp_hbm.at[:, t], xv)

        iota = lax.iota(jnp.int32, L)

        @plsc.parallel_loop(0, nj, unroll=2)
        def _(j):
            for g in range(IB // L):
                sl = (j // nj_lo, j % nj_lo, pl.ds(g * L, L))
                vv = xv[sl]
                xv[sl] = (
                    (vv & jnp.int32(-32))
                    | ((vv & 15) << 1)
                    | ((vv >> 4) & 1)
                )

        def fire(j, s):
            pltpu.async_copy(
                table_hbm.at[xv.at[j // nj_lo, j % nj_lo]],
                rows[s],
                gsems[s],
            )

        def drain_gather(s):
            pltpu.make_async_copy(
                table_hbm.at[pl.ds(0, IB)], rows[s], gsems[s]
            ).wait()

        def blk_view(p):
            return blks[p].at[:, :, pl.ds(0, IB)]

        def drain_wb(p):
            pltpu.make_async_copy(
                out_hbm.at[0, :, 0], blk_view(p), wsems[p]
            ).wait()

        def transpose(s, p):
            for kg in range(D_MODEL // L):
                kvec = iota + kg * L
                khi = kvec >> 3
                klo = kvec & 7

                @plsc.parallel_loop(0, IB, unroll=8)
                def _(i):
                    vals = rows[s][i, pl.ds(kg * L, L)]
                    col = jnp.full((L,), i, jnp.int32)
                    plsc.store_scatter(blks[p], [khi, klo, col], vals)

        for s in range(RING - 1):
            fire(s, s)

        def outer(g4, carry):
            for r in range(RING):
                j = g4 * RING + r

                @pl.when(j + RING - 1 < nj)
                def _():
                    fire(j + RING - 1, (r + RING - 1) % RING)

                drain_gather(r)
                p = r % 2

                @pl.when(j >= 2)
                def _():
                    drain_wb(p)

                transpose(r, p)
                pltpu.async_copy(blk_view(p), out_hbm.at[j, :, t], wsems[p])
            return carry

        lax.fori_loop(0, nj // RING, outer, 0)
        drain_wb(0)
        drain_wb(1)

    return kb(xp, table)


@jax.jit
def _embed(x, table):
    n_batch, n_seq = x.shape
    ni = n_batch // IB
    tab_scaled = _relayout_scale(table.T).reshape(table.shape)
    xp = (
        x.T.reshape(n_seq // 8, 8, ni, IB)
        .transpose(0, 2, 1, 3)
        .astype(jnp.int32)
    )
    out5 = _gather(xp, tab_scaled)
    return out5.transpose(2, 4, 0, 1, 3).reshape(n_batch, n_seq, D_MODEL)


def kernel(x, table):
    return _embed(x, table)


# speedup vs baseline: 1.3677x; 1.3677x over previous
import functools
import math

import jax
import jax.numpy as jnp
from jax import lax
from jax.experimental import pallas as pl
from jax.experimental.pallas import tpu as pltpu
from jax.experimental.pallas import tpu_sc as plsc

D_MODEL = 64
SCALE = math.sqrt(D_MODEL)

NC = 2
NS = 16
NW = NC * NS

IB = 128
L = 16
RING = 4


@jax.jit
def _embed(xp, table):
    nj_hi, nt, nj_lo, ib = xp.shape
    nj = nj_hi * nj_lo
    assert nt == NW and ib == IB and nj % RING == 0
    mesh = plsc.VectorSubcoreMesh(core_axis_name="c", subcore_axis_name="s")

    @functools.partial(
        pl.kernel,
        mesh=mesh,
        out_type=jax.ShapeDtypeStruct(
            (nj, D_MODEL // 8, NW, 8, IB), jnp.float32
        ),
        scratch_types=[
            pltpu.VMEM((nj_hi, nj_lo, IB), jnp.int32),
            pltpu.VMEM((IB, D_MODEL), jnp.float32),
            pltpu.VMEM((IB, D_MODEL), jnp.float32),
            pltpu.VMEM((IB, D_MODEL), jnp.float32),
            pltpu.VMEM((IB, D_MODEL), jnp.float32),
            pltpu.VMEM((D_MODEL // 8, 8, IB + 1), jnp.float32),
            pltpu.VMEM((D_MODEL // 8, 8, IB + 1), jnp.float32),
            pltpu.SemaphoreType.DMA,
            pltpu.SemaphoreType.DMA,
            pltpu.SemaphoreType.DMA,
            pltpu.SemaphoreType.DMA,
            pltpu.SemaphoreType.DMA,
            pltpu.SemaphoreType.DMA,
        ],
        compiler_params=pltpu.CompilerParams(
            use_tc_tiling_on_sc=False, needs_layout_passes=False
        ),
    )
    def k(xp_hbm, table_hbm, out_hbm,
          xv, r0, r1, r2, r3, b0, b1, g0, g1, g2, g3, w0, w1):
        t = lax.axis_index("s") * NC + lax.axis_index("c")
        rows = (r0, r1, r2, r3)
        blks = (b0, b1)
        gsems = (g0, g1, g2, g3)
        wsems = (w0, w1)

        pltpu.sync_copy(xp_hbm.at[:, t], xv)

        iota = lax.iota(jnp.int32, L)

        def fire(j, s):
            pltpu.async_copy(
                table_hbm.at[xv.at[j // nj_lo, j % nj_lo]],
                rows[s],
                gsems[s],
            )

        def drain_gather(s):
            pltpu.make_async_copy(
                table_hbm.at[pl.ds(0, IB)], rows[s], gsems[s]
            ).wait()

        def blk_view(p):
            return blks[p].at[:, :, pl.ds(0, IB)]

        def drain_wb(p):
            pltpu.make_async_copy(
                out_hbm.at[0, :, 0], blk_view(p), wsems[p]
            ).wait()

        def transpose_scale(s, p):
            for kg in range(D_MODEL // L):
                kvec = iota + kg * L
                khi = kvec >> 3
                klo = kvec & 7

                @plsc.parallel_loop(0, IB, unroll=8)
                def _(i):
                    vals = rows[s][i, pl.ds(kg * L, L)]
                    col = jnp.full((L,), i, jnp.int32)
                    plsc.store_scatter(
                        blks[p], [khi, klo, col], vals * SCALE
                    )

        for s in range(RING - 1):
            fire(s, s)

        def outer(g4, carry):
            for r in range(RING):
                j = g4 * RING + r

                @pl.when(j + RING - 1 < nj)
                def _():
                    fire(j + RING - 1, (r + RING - 1) % RING)

                drain_gather(r)
                p = r % 2

                @pl.when(j >= 2)
                def _():
                    drain_wb(p)

                transpose_scale(r, p)
                pltpu.async_copy(blk_view(p), out_hbm.at[j, :, t], wsems[p])
            return carry

        lax.fori_loop(0, nj // RING, outer, 0)
        drain_wb(0)
        drain_wb(1)

    return k(xp, table)


def kernel(x, table):
    n_batch, n_seq = x.shape
    ni = n_batch // IB
    xp = (
        x.T.reshape(n_seq // 8, 8, ni, IB)
        .transpose(0, 2, 1, 3)
        .astype(jnp.int32)
    )
    out5 = _embed(xp, table)
    return out5.transpose(2, 4, 0, 1, 3).reshape(n_batch, n_seq, D_MODEL)
